# Initial kernel scaffold; baseline (speedup 1.0000x reference)
import jax
import jax.numpy as jnp
from jax.experimental import pallas as pl


def kernel(memory, last_update, unique_node_ids, unique_messages, timestamps, W_ih, W_hh, b_ih, b_hh):
    raise NotImplementedError("write your pallas kernel here")


# speedup vs baseline: 1.9736x; 1.9736x over previous
import functools

import jax
import jax.numpy as jnp
from jax import lax
from jax.experimental import pallas as pl
from jax.experimental.pallas import tpu as pltpu
from jax.experimental.pallas import tpu_sc as plsc

NC = 2
NS = 16
NW = NC * NS
LANES = 16


def _wid():
    return lax.axis_index("s") * NC + lax.axis_index("c")


@functools.lru_cache(maxsize=None)
def _gather_rows(B, M, D):
    b_per_w = B // NW
    mesh = plsc.VectorSubcoreMesh(core_axis_name="c", subcore_axis_name="s")

    @functools.partial(
        pl.kernel,
        mesh=mesh,
        out_type=jax.ShapeDtypeStruct((B, D), jnp.float32),
        scratch_types=[
            pltpu.VMEM((b_per_w,), jnp.int32),
            pltpu.VMEM((b_per_w, D), jnp.float32),
            pltpu.SemaphoreType.DMA,
        ],
    )
    def gather_k(ids_hbm, table_hbm, out_hbm, idx_v, rows_v, sem):
        base = _wid() * b_per_w
        pltpu.sync_copy(ids_hbm.at[pl.ds(base, b_per_w)], idx_v)
        pltpu.async_copy(table_hbm.at[idx_v], rows_v, sem).wait()
        pltpu.sync_copy(rows_v, out_hbm.at[pl.ds(base, b_per_w)])

    return gather_k


@functools.lru_cache(maxsize=None)
def _gru(B, D, BLK=2048):
    grid = (B // BLK,)

    def body(msg_ref, h_ref, wih_ref, whh_ref, bih_ref, bhh_ref, out_ref):
        gi = jnp.dot(msg_ref[...], wih_ref[...],
                     preferred_element_type=jnp.float32) + bih_ref[...]
        gh = jnp.dot(h_ref[...], whh_ref[...],
                     preferred_element_type=jnp.float32) + bhh_ref[...]
        i_r, i_z, i_n = gi[:, :D], gi[:, D:2 * D], gi[:, 2 * D:]
        h_r, h_z, h_n = gh[:, :D], gh[:, D:2 * D], gh[:, 2 * D:]
        r = jax.nn.sigmoid(i_r + h_r)
        z = jax.nn.sigmoid(i_z + h_z)
        n = jnp.tanh(i_n + r * h_n)
        h = h_ref[...]
        out_ref[...] = (1.0 - z) * n + z * h

    return pl.pallas_call(
        body,
        grid=grid,
        in_specs=[
            pl.BlockSpec((BLK, D), lambda i: (i, 0)),
            pl.BlockSpec((BLK, D), lambda i: (i, 0)),
            pl.BlockSpec((D, 3 * D), lambda i: (0, 0)),
            pl.BlockSpec((D, 3 * D), lambda i: (0, 0)),
            pl.BlockSpec((1, 3 * D), lambda i: (0, 0)),
            pl.BlockSpec((1, 3 * D), lambda i: (0, 0)),
        ],
        out_specs=pl.BlockSpec((BLK, D), lambda i: (i, 0)),
        out_shape=jax.ShapeDtypeStruct((B, D), jnp.float32),
    )


@functools.lru_cache(maxsize=None)
def _scatter(B, M, D):
    b_per_w = B // NW
    n_chunks = B // LANES
    rng = (-(-M // NS) + 7) // 8 * 8
    m_pad = rng * NS
    mesh = plsc.VectorSubcoreMesh(core_axis_name="c", subcore_axis_name="s")

    @functools.partial(
        pl.kernel,
        mesh=mesh,
        out_type=(),
        compiler_params=pltpu.CompilerParams(needs_layout_passes=False),
        scratch_types=[
            pltpu.VMEM((B,), jnp.int32),
            pltpu.VMEM((rng,), jnp.int32),
            pltpu.VMEM((b_per_w,), jnp.int32),
            pltpu.VMEM((b_per_w,), jnp.int32),
            pltpu.VMEM((b_per_w, D), jnp.float32),
            pltpu.VMEM((b_per_w,), jnp.float32),
            pltpu.VMEM_SHARED((m_pad,), jnp.int32),
            pltpu.SemaphoreType.DMA,
            pltpu.SemaphoreType.DMA,
        ],
    )
    def scatter_k(ids_hbm, rows_hbm, ts_hbm, mem_ref, lu_ref,
                  ids_all, mp_loc, idx_v, mp_v, rows_v, ts_v, mp_sh,
                  sem0, sem1):
        s = lax.axis_index("s")
        wid = _wid()
        pltpu.sync_copy(ids_hbm, ids_all)
        lo = s * rng
        lane = lax.iota(jnp.int32, LANES)

        def body(k, carry):
            idx = ids_all[pl.ds(k * LANES, LANES)]
            pos = jnp.full((LANES,), k * LANES, jnp.int32) + lane
            _, last_m = plsc.scan_count(idx)
            m = (idx >= lo) & (idx < lo + rng) & last_m
            loc = jnp.clip(idx - lo, 0, rng - 1)
            plsc.store_scatter(mp_loc, [loc], pos, mask=m)
            return carry

        lax.fori_loop(0, n_chunks, body, 0)
        pltpu.sync_copy(mp_loc, mp_sh.at[pl.ds(lo, rng)])
        plsc.subcore_barrier()

        base = wid * b_per_w
        pltpu.sync_copy(ids_hbm.at[pl.ds(base, b_per_w)], idx_v)
        pltpu.async_copy(mp_sh.at[idx_v], mp_v, sem0).wait()
        pltpu.async_copy(rows_hbm.at[mp_v], rows_v, sem0).wait()
        pltpu.async_copy(ts_hbm.at[mp_v], ts_v, sem1).wait()
        pltpu.async_copy(rows_v, mem_ref.at[idx_v], sem0).wait()
        pltpu.async_copy(ts_v, lu_ref.at[idx_v], sem1).wait()

    return scatter_k


def kernel(memory, last_update, unique_node_ids, unique_messages, timestamps,
           W_ih, W_hh, b_ih, b_hh):
    M, D = memory.shape
    B = unique_node_ids.shape[0]
    ids = unique_node_ids.astype(jnp.int32)

    h = _gather_rows(B, M, D)(ids, memory)
    rows = _gru(B, D)(unique_messages, h, W_ih.T, W_hh.T,
                      b_ih.reshape(1, -1), b_hh.reshape(1, -1))

    mem_ref = jax.new_ref(memory)
    lu_ref = jax.new_ref(last_update)
    _scatter(B, M, D)(ids, rows, timestamps, mem_ref, lu_ref)
    return mem_ref[...], lu_ref[...]


# speedup vs baseline: 2.0402x; 1.0337x over previous
import functools

import jax
import jax.numpy as jnp
from jax import lax
from jax.experimental import pallas as pl
from jax.experimental.pallas import tpu as pltpu
from jax.experimental.pallas import tpu_sc as plsc

NC = 2
NS = 16
NW = NC * NS
LANES = 16


def _wid():
    return lax.axis_index("s") * NC + lax.axis_index("c")


@functools.lru_cache(maxsize=None)
def _gather_rows(B, M, D):
    b_per_w = B // NW
    mesh = plsc.VectorSubcoreMesh(core_axis_name="c", subcore_axis_name="s")

    @functools.partial(
        pl.kernel,
        mesh=mesh,
        out_type=jax.ShapeDtypeStruct((B, D), jnp.float32),
        scratch_types=[
            pltpu.VMEM((b_per_w,), jnp.int32),
            pltpu.VMEM((b_per_w, D), jnp.float32),
            pltpu.SemaphoreType.DMA,
        ],
    )
    def gather_k(ids_hbm, table_hbm, out_hbm, idx_v, rows_v, sem):
        base = _wid() * b_per_w
        pltpu.sync_copy(ids_hbm.at[pl.ds(base, b_per_w)], idx_v)
        pltpu.async_copy(table_hbm.at[idx_v], rows_v, sem).wait()
        pltpu.sync_copy(rows_v, out_hbm.at[pl.ds(base, b_per_w)])

    return gather_k


@functools.lru_cache(maxsize=None)
def _gru(B, D, BLK=2048):
    grid = (B // BLK,)

    def body(msg_ref, h_ref, wih_ref, whh_ref, bih_ref, bhh_ref, out_ref):
        gi = jnp.dot(msg_ref[...], wih_ref[...],
                     preferred_element_type=jnp.float32) + bih_ref[...]
        gh = jnp.dot(h_ref[...], whh_ref[...],
                     preferred_element_type=jnp.float32) + bhh_ref[...]
        i_r, i_z, i_n = gi[:, :D], gi[:, D:2 * D], gi[:, 2 * D:]
        h_r, h_z, h_n = gh[:, :D], gh[:, D:2 * D], gh[:, 2 * D:]
        r = jax.nn.sigmoid(i_r + h_r)
        z = jax.nn.sigmoid(i_z + h_z)
        n = jnp.tanh(i_n + r * h_n)
        h = h_ref[...]
        out_ref[...] = (1.0 - z) * n + z * h

    return pl.pallas_call(
        body,
        grid=grid,
        in_specs=[
            pl.BlockSpec((BLK, D), lambda i: (i, 0)),
            pl.BlockSpec((BLK, D), lambda i: (i, 0)),
            pl.BlockSpec((D, 3 * D), lambda i: (0, 0)),
            pl.BlockSpec((D, 3 * D), lambda i: (0, 0)),
            pl.BlockSpec((1, 3 * D), lambda i: (0, 0)),
            pl.BlockSpec((1, 3 * D), lambda i: (0, 0)),
        ],
        out_specs=pl.BlockSpec((BLK, D), lambda i: (i, 0)),
        out_shape=jax.ShapeDtypeStruct((B, D), jnp.float32),
    )


@functools.lru_cache(maxsize=None)
def _scatter(B, M, D, NB=4, UNROLL=4):
    b_per_w = B // NW
    blk = b_per_w // NB
    n_chunks = B // LANES
    rng = (-(-M // NS) + 7) // 8 * 8
    m_pad = rng * NS
    mesh = plsc.VectorSubcoreMesh(core_axis_name="c", subcore_axis_name="s")

    @functools.partial(
        pl.kernel,
        mesh=mesh,
        out_type=(),
        compiler_params=pltpu.CompilerParams(needs_layout_passes=False),
        scratch_types=[
            pltpu.VMEM((B,), jnp.int32),
            pltpu.VMEM((rng,), jnp.int32),
            pltpu.VMEM((NB, blk), jnp.int32),
            pltpu.VMEM((b_per_w,), jnp.int32),
            pltpu.VMEM((NB, blk, D), jnp.float32),
            pltpu.VMEM((NB, blk), jnp.float32),
            pltpu.VMEM_SHARED((m_pad,), jnp.int32),
            pltpu.SemaphoreType.DMA,
            [pltpu.SemaphoreType.DMA] * NB,
            [pltpu.SemaphoreType.DMA] * NB,
        ],
    )
    def scatter_k(ids_hbm, rows_hbm, ts_hbm, mem_ref, lu_ref,
                  ids_all, mp_loc, idx_b, mp_v, rows_b, ts_b, mp_sh,
                  sem0, gsems, ssems):
        s = lax.axis_index("s")
        wid = _wid()
        stage = [pltpu.async_copy(
            ids_hbm.at[pl.ds(wid * b_per_w + j * blk, blk)],
            idx_b.at[j], gsems[j]) for j in range(NB)]

        pltpu.sync_copy(ids_hbm, ids_all)
        lo = s * rng
        lane = lax.iota(jnp.int32, LANES)

        def chunk(k):
            idx = ids_all[pl.ds(k * LANES, LANES)]
            pos = jnp.full((LANES,), k * LANES, jnp.int32) + lane
            _, last_m = plsc.scan_count(idx)
            m = (idx >= lo) & (idx < lo + rng) & last_m
            loc = jnp.clip(idx - lo, 0, rng - 1)
            plsc.store_scatter(mp_loc, [loc], pos, mask=m)

        def body(k, carry):
            for u in range(UNROLL):
                chunk(k * UNROLL + u)
            return carry

        lax.fori_loop(0, n_chunks // UNROLL, body, 0)
        pltpu.sync_copy(mp_loc, mp_sh.at[pl.ds(lo, rng)])
        for cp in stage:
            cp.wait()
        plsc.subcore_barrier()

        for j in range(NB):
            pltpu.async_copy(mp_sh.at[idx_b.at[j]],
                             mp_v.at[pl.ds(j * blk, blk)], sem0).wait()
        row_g, ts_g = [], []
        for j in range(NB):
            row_g.append(pltpu.async_copy(
                rows_hbm.at[mp_v.at[pl.ds(j * blk, blk)]],
                rows_b.at[j], gsems[j]))
            ts_g.append(pltpu.async_copy(
                ts_hbm.at[mp_v.at[pl.ds(j * blk, blk)]],
                ts_b.at[j], ssems[j]))
        row_s = []
        for j in range(NB):
            row_g[j].wait()
            ts_g[j].wait()
            row_s.append(pltpu.async_copy(
                rows_b.at[j], mem_ref.at[idx_b.at[j]], gsems[j]))
            row_s.append(pltpu.async_copy(
                ts_b.at[j], lu_ref.at[idx_b.at[j]], ssems[j]))
        for cp in row_s:
            cp.wait()

    return scatter_k


def kernel(memory, last_update, unique_node_ids, unique_messages, timestamps,
           W_ih, W_hh, b_ih, b_hh):
    M, D = memory.shape
    B = unique_node_ids.shape[0]
    ids = unique_node_ids.astype(jnp.int32)

    h = _gather_rows(B, M, D)(ids, memory)
    rows = _gru(B, D)(unique_messages, h, W_ih.T, W_hh.T,
                      b_ih.reshape(1, -1), b_hh.reshape(1, -1))

    mem_ref = jax.new_ref(memory)
    lu_ref = jax.new_ref(last_update)
    _scatter(B, M, D)(ids, rows, timestamps, mem_ref, lu_ref)
    return mem_ref[...], lu_ref[...]


# speedup vs baseline: 2.2255x; 1.0908x over previous
import functools

import jax
import jax.numpy as jnp
from jax import lax
from jax.experimental import pallas as pl
from jax.experimental.pallas import tpu as pltpu
from jax.experimental.pallas import tpu_sc as plsc

NC = 2
NS = 16
NW = NC * NS
LANES = 16


def _wid():
    return lax.axis_index("s") * NC + lax.axis_index("c")


@functools.lru_cache(maxsize=None)
def _gather_rows(B, M, D):
    b_per_w = B // NW
    mesh = plsc.VectorSubcoreMesh(core_axis_name="c", subcore_axis_name="s")

    @functools.partial(
        pl.kernel,
        mesh=mesh,
        out_type=jax.ShapeDtypeStruct((B, D), jnp.float32),
        scratch_types=[
            pltpu.VMEM((b_per_w,), jnp.int32),
            pltpu.VMEM((b_per_w, D), jnp.float32),
            pltpu.SemaphoreType.DMA,
        ],
    )
    def gather_k(ids_hbm, table_hbm, out_hbm, idx_v, rows_v, sem):
        base = _wid() * b_per_w
        pltpu.sync_copy(ids_hbm.at[pl.ds(base, b_per_w)], idx_v)
        pltpu.async_copy(table_hbm.at[idx_v], rows_v, sem).wait()
        pltpu.sync_copy(rows_v, out_hbm.at[pl.ds(base, b_per_w)])

    return gather_k


@functools.lru_cache(maxsize=None)
def _gru(B, D, BLK=2048):
    grid = (B // BLK,)

    def body(msg_ref, h_ref, wih_ref, whh_ref, bih_ref, bhh_ref, out_ref):
        gi = jnp.dot(msg_ref[...], wih_ref[...],
                     preferred_element_type=jnp.float32) + bih_ref[...]
        gh = jnp.dot(h_ref[...], whh_ref[...],
                     preferred_element_type=jnp.float32) + bhh_ref[...]
        i_r, i_z, i_n = gi[:, :D], gi[:, D:2 * D], gi[:, 2 * D:]
        h_r, h_z, h_n = gh[:, :D], gh[:, D:2 * D], gh[:, 2 * D:]
        r = jax.nn.sigmoid(i_r + h_r)
        z = jax.nn.sigmoid(i_z + h_z)
        n = jnp.tanh(i_n + r * h_n)
        h = h_ref[...]
        out_ref[...] = (1.0 - z) * n + z * h

    return pl.pallas_call(
        body,
        grid=grid,
        in_specs=[
            pl.BlockSpec((BLK, D), lambda i: (i, 0)),
            pl.BlockSpec((BLK, D), lambda i: (i, 0)),
            pl.BlockSpec((D, 3 * D), lambda i: (0, 0)),
            pl.BlockSpec((D, 3 * D), lambda i: (0, 0)),
            pl.BlockSpec((1, 3 * D), lambda i: (0, 0)),
            pl.BlockSpec((1, 3 * D), lambda i: (0, 0)),
        ],
        out_specs=pl.BlockSpec((BLK, D), lambda i: (i, 0)),
        out_shape=jax.ShapeDtypeStruct((B, D), jnp.float32),
    )


@functools.lru_cache(maxsize=None)
def _scatter(B, M, D, NB=4, UNROLL=4):
    b_per_w = B // NW
    blk = b_per_w // NB
    n_chunks = B // LANES
    rng = (-(-M // NS) + 7) // 8 * 8
    m_pad = rng * NS
    mesh = plsc.VectorSubcoreMesh(core_axis_name="c", subcore_axis_name="s")

    @functools.partial(
        pl.kernel,
        mesh=mesh,
        out_type=(),
        compiler_params=pltpu.CompilerParams(needs_layout_passes=False),
        scratch_types=[
            pltpu.VMEM((B,), jnp.int32),
            pltpu.VMEM((rng,), jnp.int32),
            pltpu.VMEM((NB, blk), jnp.int32),
            pltpu.VMEM((b_per_w,), jnp.int32),
            pltpu.VMEM((NB, blk, D), jnp.float32),
            pltpu.VMEM((NB, blk), jnp.float32),
            pltpu.VMEM_SHARED((m_pad,), jnp.int32),
            pltpu.SemaphoreType.DMA,
            [pltpu.SemaphoreType.DMA] * NB,
            [pltpu.SemaphoreType.DMA] * NB,
        ],
    )
    def scatter_k(ids_hbm, rows_hbm, ts_hbm, mem_ref, lu_ref,
                  ids_all, mp_loc, idx_b, mp_v, rows_b, ts_b, mp_sh,
                  sem0, gsems, ssems):
        s = lax.axis_index("s")
        wid = _wid()
        stage = [pltpu.async_copy(
            ids_hbm.at[pl.ds(wid * b_per_w + j * blk, blk)],
            idx_b.at[j], gsems[j]) for j in range(NB)]

        pltpu.sync_copy(ids_hbm, ids_all)
        lo = s * rng
        lane = lax.iota(jnp.int32, LANES)

        def body(k, carry):
            ks = [k * UNROLL + u for u in range(UNROLL)]
            idxs = [ids_all[pl.ds(kk * LANES, LANES)] for kk in ks]
            lasts = [plsc.scan_count(idx)[1] for idx in idxs]
            for kk, idx, last_m in zip(ks, idxs, lasts):
                pos = jnp.full((LANES,), kk * LANES, jnp.int32) + lane
                m = (idx >= lo) & (idx < lo + rng) & last_m
                loc = jnp.clip(idx - lo, 0, rng - 1)
                plsc.store_scatter(mp_loc, [loc], pos, mask=m)
            return carry

        lax.fori_loop(0, n_chunks // UNROLL, body, 0)
        pltpu.sync_copy(mp_loc, mp_sh.at[pl.ds(lo, rng)])
        for cp in stage:
            cp.wait()
        plsc.subcore_barrier()

        for j in range(NB):
            pltpu.async_copy(mp_sh.at[idx_b.at[j]],
                             mp_v.at[pl.ds(j * blk, blk)], sem0).wait()
        row_g, ts_g = [], []
        for j in range(NB):
            row_g.append(pltpu.async_copy(
                rows_hbm.at[mp_v.at[pl.ds(j * blk, blk)]],
                rows_b.at[j], gsems[j]))
            ts_g.append(pltpu.async_copy(
                ts_hbm.at[mp_v.at[pl.ds(j * blk, blk)]],
                ts_b.at[j], ssems[j]))
        row_s = []
        for j in range(NB):
            row_g[j].wait()
            ts_g[j].wait()
            row_s.append(pltpu.async_copy(
                rows_b.at[j], mem_ref.at[idx_b.at[j]], gsems[j]))
            row_s.append(pltpu.async_copy(
                ts_b.at[j], lu_ref.at[idx_b.at[j]], ssems[j]))
        for cp in row_s:
            cp.wait()

    return scatter_k


def kernel(memory, last_update, unique_node_ids, unique_messages, timestamps,
           W_ih, W_hh, b_ih, b_hh):
    M, D = memory.shape
    B = unique_node_ids.shape[0]
    ids = unique_node_ids.astype(jnp.int32)

    h = _gather_rows(B, M, D)(ids, memory)
    rows = _gru(B, D)(unique_messages, h, W_ih.T, W_hh.T,
                      b_ih.reshape(1, -1), b_hh.reshape(1, -1))

    mem_ref = jax.new_ref(memory)
    lu_ref = jax.new_ref(last_update)
    _scatter(B, M, D)(ids, rows, timestamps, mem_ref, lu_ref)
    return mem_ref[...], lu_ref[...]


# speedup vs baseline: 2.3123x; 1.0390x over previous
import functools

import jax
import jax.numpy as jnp
from jax import lax
from jax.experimental import pallas as pl
from jax.experimental.pallas import tpu as pltpu
from jax.experimental.pallas import tpu_sc as plsc

NC = 2
NS = 16
NW = NC * NS
LANES = 16


def _wid():
    return lax.axis_index("s") * NC + lax.axis_index("c")


def _rng(M):
    return (-(-M // NS) + 7) // 8 * 8


@functools.lru_cache(maxsize=None)
def _gather_maxpos(B, M, D, UNROLL=4):
    b_per_w = B // NW
    n_chunks = B // LANES
    rng = _rng(M)
    m_pad = rng * NS
    mesh = plsc.VectorSubcoreMesh(core_axis_name="c", subcore_axis_name="s")

    @functools.partial(
        pl.kernel,
        mesh=mesh,
        out_type=(
            jax.ShapeDtypeStruct((B, D), jnp.float32),
            jax.ShapeDtypeStruct((m_pad,), jnp.int32),
        ),
        compiler_params=pltpu.CompilerParams(needs_layout_passes=False),
        scratch_types=[
            pltpu.VMEM((B,), jnp.int32),
            pltpu.VMEM((rng,), jnp.int32),
            pltpu.VMEM((b_per_w, D), jnp.float32),
            pltpu.SemaphoreType.DMA,
        ],
    )
    def gather_k(ids_hbm, table_hbm, h_hbm, mp_hbm, ids_all, mp_loc, rows_v,
                 sem):
        s = lax.axis_index("s")
        wid = _wid()
        base = wid * b_per_w
        pltpu.sync_copy(ids_hbm, ids_all)
        g = pltpu.async_copy(
            table_hbm.at[ids_all.at[pl.ds(base, b_per_w)]], rows_v, sem)

        lo = s * rng
        lane = lax.iota(jnp.int32, LANES)

        def body(k, carry):
            ks = [k * UNROLL + u for u in range(UNROLL)]
            idxs = [ids_all[pl.ds(kk * LANES, LANES)] for kk in ks]
            lasts = [plsc.scan_count(idx)[1] for idx in idxs]
            for kk, idx, last_m in zip(ks, idxs, lasts):
                pos = jnp.full((LANES,), kk * LANES, jnp.int32) + lane
                m = (idx >= lo) & (idx < lo + rng) & last_m
                loc = jnp.clip(idx - lo, 0, rng - 1)
                plsc.store_scatter(mp_loc, [loc], pos, mask=m)
            return carry

        lax.fori_loop(0, n_chunks // UNROLL, body, 0)
        pltpu.sync_copy(mp_loc, mp_hbm.at[pl.ds(lo, rng)])
        g.wait()
        pltpu.sync_copy(rows_v, h_hbm.at[pl.ds(base, b_per_w)])

    return gather_k


@functools.lru_cache(maxsize=None)
def _gru_copy(B, M, D, BLK=1024):
    grid = (B // BLK,)
    cpb = -(-M // (B // BLK))
    cpb = (cpb + 7) // 8 * 8

    def body(msg_ref, h_ref, wih_ref, whh_ref, bih_ref, bhh_ref, mem_ref,
             rows_ref, cp_ref):
        gi = jnp.dot(msg_ref[...], wih_ref[...],
                     preferred_element_type=jnp.float32) + bih_ref[...]
        gh = jnp.dot(h_ref[...], whh_ref[...],
                     preferred_element_type=jnp.float32) + bhh_ref[...]
        i_r, i_z, i_n = gi[:, :D], gi[:, D:2 * D], gi[:, 2 * D:]
        h_r, h_z, h_n = gh[:, :D], gh[:, D:2 * D], gh[:, 2 * D:]
        r = jax.nn.sigmoid(i_r + h_r)
        z = jax.nn.sigmoid(i_z + h_z)
        n = jnp.tanh(i_n + r * h_n)
        h = h_ref[...]
        rows_ref[...] = (1.0 - z) * n + z * h
        cp_ref[...] = mem_ref[...]

    return pl.pallas_call(
        body,
        grid=grid,
        in_specs=[
            pl.BlockSpec((BLK, D), lambda i: (i, 0)),
            pl.BlockSpec((BLK, D), lambda i: (i, 0)),
            pl.BlockSpec((D, 3 * D), lambda i: (0, 0)),
            pl.BlockSpec((D, 3 * D), lambda i: (0, 0)),
            pl.BlockSpec((1, 3 * D), lambda i: (0, 0)),
            pl.BlockSpec((1, 3 * D), lambda i: (0, 0)),
            pl.BlockSpec((cpb, D), lambda i: (i, 0)),
        ],
        out_specs=[
            pl.BlockSpec((BLK, D), lambda i: (i, 0)),
            pl.BlockSpec((cpb, D), lambda i: (i, 0)),
        ],
        out_shape=[
            jax.ShapeDtypeStruct((B, D), jnp.float32),
            jax.ShapeDtypeStruct((M, D), jnp.float32),
        ],
    )


@functools.lru_cache(maxsize=None)
def _scatter(B, M, D, NB=4):
    b_per_w = B // NW
    blk = b_per_w // NB
    rng = _rng(M)
    mesh = plsc.VectorSubcoreMesh(core_axis_name="c", subcore_axis_name="s")

    @functools.partial(
        pl.kernel,
        mesh=mesh,
        out_type=(),
        compiler_params=pltpu.CompilerParams(needs_layout_passes=False),
        scratch_types=[
            pltpu.VMEM((NB, blk), jnp.int32),
            pltpu.VMEM((NB, blk), jnp.int32),
            pltpu.VMEM((NB, blk, D), jnp.float32),
            pltpu.VMEM((NB, blk), jnp.float32),
            pltpu.SemaphoreType.DMA,
            [pltpu.SemaphoreType.DMA] * NB,
            [pltpu.SemaphoreType.DMA] * NB,
            [pltpu.SemaphoreType.DMA] * NB,
        ],
    )
    def scatter_k(ids_hbm, mp_hbm, rows_hbm, ts_hbm, mem_ref, lu_ref,
                  idx_b, mp_b, rows_b, ts_b, sem0, gsems, ssems, tsems):
        wid = _wid()
        base = wid * b_per_w
        for j in range(NB):
            pltpu.sync_copy(ids_hbm.at[pl.ds(base + j * blk, blk)],
                            idx_b.at[j])
        mp_g = [pltpu.async_copy(mp_hbm.at[idx_b.at[j]], mp_b.at[j], gsems[j])
                for j in range(NB)]
        row_g, ts_g, outs = [], [], []
        for j in range(NB):
            mp_g[j].wait()
            row_g.append(pltpu.async_copy(
                rows_hbm.at[mp_b.at[j]], rows_b.at[j], gsems[j]))
            ts_g.append(pltpu.async_copy(
                ts_hbm.at[mp_b.at[j]], ts_b.at[j], tsems[j]))
        for j in range(NB):
            row_g[j].wait()
            outs.append(pltpu.async_copy(
                rows_b.at[j], mem_ref.at[idx_b.at[j]], ssems[j]))
            ts_g[j].wait()
            outs.append(pltpu.async_copy(
                ts_b.at[j], lu_ref.at[idx_b.at[j]], tsems[j]))
        for cp in outs:
            cp.wait()

    return scatter_k


def kernel(memory, last_update, unique_node_ids, unique_messages, timestamps,
           W_ih, W_hh, b_ih, b_hh):
    M, D = memory.shape
    B = unique_node_ids.shape[0]
    ids = unique_node_ids.astype(jnp.int32)

    h, maxpos = _gather_maxpos(B, M, D)(ids, memory)
    rows, mem_copy = _gru_copy(B, M, D)(
        unique_messages, h, W_ih.T, W_hh.T,
        b_ih.reshape(1, -1), b_hh.reshape(1, -1), memory)

    mem_ref = jax.new_ref(mem_copy)
    lu_ref = jax.new_ref(last_update)
    _scatter(B, M, D)(ids, maxpos, rows, timestamps, mem_ref, lu_ref)
    return mem_ref[...], lu_ref[...]


# speedup vs baseline: 2.6131x; 1.1301x over previous
import functools

import jax
import jax.numpy as jnp
from jax import lax
from jax.experimental import pallas as pl
from jax.experimental.pallas import tpu as pltpu
from jax.experimental.pallas import tpu_sc as plsc

NC = 2
NS = 16
NW = NC * NS
LANES = 16


def _wid():
    return lax.axis_index("s") * NC + lax.axis_index("c")


def _rng(M):
    return (-(-M // NS) + 7) // 8 * 8


@functools.lru_cache(maxsize=None)
def _gather_maxpos(B, M, D, UNROLL=4):
    b_per_w = B // NW
    n_chunks = B // LANES
    rng = _rng(M)
    m_pad = rng * NS
    mesh = plsc.VectorSubcoreMesh(core_axis_name="c", subcore_axis_name="s")

    @functools.partial(
        pl.kernel,
        mesh=mesh,
        out_type=(
            jax.ShapeDtypeStruct((B, D), jnp.float32),
            jax.ShapeDtypeStruct((m_pad,), jnp.int32),
            jax.ShapeDtypeStruct((m_pad,), jnp.float32),
        ),
        compiler_params=pltpu.CompilerParams(needs_layout_passes=False),
        scratch_types=[
            pltpu.VMEM((B,), jnp.int32),
            pltpu.VMEM((B,), jnp.float32),
            pltpu.VMEM((rng,), jnp.int32),
            pltpu.VMEM((rng,), jnp.float32),
            pltpu.VMEM((rng,), jnp.float32),
            pltpu.VMEM((b_per_w, D), jnp.float32),
            pltpu.SemaphoreType.DMA,
            pltpu.SemaphoreType.DMA,
        ],
    )
    def gather_k(ids_hbm, ts_hbm, lu_hbm, table_hbm, h_hbm, mp_hbm, luo_hbm,
                 ids_all, ts_all, mp_loc, ts_loc, lu_v, rows_v, sem, sem2):
        s = lax.axis_index("s")
        wid = _wid()
        base = wid * b_per_w
        lo = s * rng
        pltpu.sync_copy(ids_hbm, ids_all)
        g = pltpu.async_copy(
            table_hbm.at[ids_all.at[pl.ds(base, b_per_w)]], rows_v, sem)
        g2 = pltpu.async_copy(ts_hbm, ts_all, sem2)
        g3 = pltpu.async_copy(lu_hbm.at[pl.ds(lo, rng)], lu_v, sem2)

        lane = lax.iota(jnp.int32, LANES)
        neg1 = jnp.full((LANES,), -1, jnp.int32)

        def clr(r, carry):
            mp_loc[pl.ds(r * LANES, LANES)] = neg1
            return carry

        lax.fori_loop(0, rng // LANES, clr, 0)
        g2.wait()

        def body(k, carry):
            ks = [k * UNROLL + u for u in range(UNROLL)]
            idxs = [ids_all[pl.ds(kk * LANES, LANES)] for kk in ks]
            tss = [ts_all[pl.ds(kk * LANES, LANES)] for kk in ks]
            lasts = [plsc.scan_count(idx)[1] for idx in idxs]
            for kk, idx, ts, last_m in zip(ks, idxs, tss, lasts):
                pos = jnp.full((LANES,), kk * LANES, jnp.int32) + lane
                m = (idx >= lo) & (idx < lo + rng) & last_m
                loc = jnp.clip(idx - lo, 0, rng - 1)
                plsc.store_scatter(mp_loc, [loc], pos, mask=m)
                plsc.store_scatter(ts_loc, [loc], ts, mask=m)
            return carry

        lax.fori_loop(0, n_chunks // UNROLL, body, 0)
        pltpu.sync_copy(mp_loc, mp_hbm.at[pl.ds(lo, rng)])

        g3.wait()

        def mrg(r, carry):
            sl = pl.ds(r * LANES, LANES)
            touched = mp_loc[sl] >= 0
            lu_v[sl] = jnp.where(touched, ts_loc[sl], lu_v[sl])
            return carry

        lax.fori_loop(0, rng // LANES, mrg, 0)
        pltpu.sync_copy(lu_v, luo_hbm.at[pl.ds(lo, rng)])
        g.wait()
        pltpu.sync_copy(rows_v, h_hbm.at[pl.ds(base, b_per_w)])

    return gather_k


@functools.lru_cache(maxsize=None)
def _gru_copy(B, M, D, BLK=1024):
    grid = (B // BLK,)
    cpb = -(-M // (B // BLK))
    cpb = (cpb + 7) // 8 * 8

    def body(msg_ref, h_ref, wih_ref, whh_ref, bih_ref, bhh_ref, mem_ref,
             rows_ref, cp_ref):
        gi = jnp.dot(msg_ref[...], wih_ref[...],
                     preferred_element_type=jnp.float32) + bih_ref[...]
        gh = jnp.dot(h_ref[...], whh_ref[...],
                     preferred_element_type=jnp.float32) + bhh_ref[...]
        i_r, i_z, i_n = gi[:, :D], gi[:, D:2 * D], gi[:, 2 * D:]
        h_r, h_z, h_n = gh[:, :D], gh[:, D:2 * D], gh[:, 2 * D:]
        r = jax.nn.sigmoid(i_r + h_r)
        z = jax.nn.sigmoid(i_z + h_z)
        n = jnp.tanh(i_n + r * h_n)
        h = h_ref[...]
        rows_ref[...] = (1.0 - z) * n + z * h
        cp_ref[...] = mem_ref[...]

    return pl.pallas_call(
        body,
        grid=grid,
        in_specs=[
            pl.BlockSpec((BLK, D), lambda i: (i, 0)),
            pl.BlockSpec((BLK, D), lambda i: (i, 0)),
            pl.BlockSpec((D, 3 * D), lambda i: (0, 0)),
            pl.BlockSpec((D, 3 * D), lambda i: (0, 0)),
            pl.BlockSpec((1, 3 * D), lambda i: (0, 0)),
            pl.BlockSpec((1, 3 * D), lambda i: (0, 0)),
            pl.BlockSpec((cpb, D), lambda i: (i, 0)),
        ],
        out_specs=[
            pl.BlockSpec((BLK, D), lambda i: (i, 0)),
            pl.BlockSpec((cpb, D), lambda i: (i, 0)),
        ],
        out_shape=[
            jax.ShapeDtypeStruct((B, D), jnp.float32),
            jax.ShapeDtypeStruct((M, D), jnp.float32),
        ],
    )


@functools.lru_cache(maxsize=None)
def _scatter(B, M, D, NB=4):
    b_per_w = B // NW
    blk = b_per_w // NB
    rng = _rng(M)
    mesh = plsc.VectorSubcoreMesh(core_axis_name="c", subcore_axis_name="s")

    @functools.partial(
        pl.kernel,
        mesh=mesh,
        out_type=(),
        compiler_params=pltpu.CompilerParams(needs_layout_passes=False),
        scratch_types=[
            pltpu.VMEM((NB, blk), jnp.int32),
            pltpu.VMEM((NB, blk), jnp.int32),
            pltpu.VMEM((NB, blk, D), jnp.float32),
            [pltpu.SemaphoreType.DMA] * NB,
            [pltpu.SemaphoreType.DMA] * NB,
        ],
    )
    def scatter_k(ids_hbm, mp_hbm, rows_hbm, mem_ref,
                  idx_b, mp_b, rows_b, gsems, ssems):
        wid = _wid()
        base = wid * b_per_w
        for j in range(NB):
            pltpu.sync_copy(ids_hbm.at[pl.ds(base + j * blk, blk)],
                            idx_b.at[j])
        mp_g = [pltpu.async_copy(mp_hbm.at[idx_b.at[j]], mp_b.at[j], gsems[j])
                for j in range(NB)]
        row_g, outs = [], []
        for j in range(NB):
            mp_g[j].wait()
            row_g.append(pltpu.async_copy(
                rows_hbm.at[mp_b.at[j]], rows_b.at[j], gsems[j]))
        for j in range(NB):
            row_g[j].wait()
            outs.append(pltpu.async_copy(
                rows_b.at[j], mem_ref.at[idx_b.at[j]], ssems[j]))
        for cp in outs:
            cp.wait()

    return scatter_k


def kernel(memory, last_update, unique_node_ids, unique_messages, timestamps,
           W_ih, W_hh, b_ih, b_hh):
    M, D = memory.shape
    B = unique_node_ids.shape[0]
    ids = unique_node_ids.astype(jnp.int32)

    m_pad = _rng(M) * NS
    lu_pad = jnp.pad(last_update, (0, m_pad - M))
    h, maxpos, lu_full = _gather_maxpos(B, M, D)(ids, timestamps, lu_pad,
                                                 memory)
    rows, mem_copy = _gru_copy(B, M, D)(
        unique_messages, h, W_ih.T, W_hh.T,
        b_ih.reshape(1, -1), b_hh.reshape(1, -1), memory)

    mem_ref = jax.new_ref(mem_copy)
    _scatter(B, M, D)(ids, maxpos, rows, mem_ref)
    return mem_ref[...], lu_full[:M]


# speedup vs baseline: 2.6874x; 1.0284x over previous
import functools

import jax
import jax.numpy as jnp
from jax import lax
from jax.experimental import pallas as pl
from jax.experimental.pallas import tpu as pltpu
from jax.experimental.pallas import tpu_sc as plsc

NC = 2
NS = 16
NW = NC * NS
LANES = 16


def _wid():
    return lax.axis_index("s") * NC + lax.axis_index("c")


def _rng(M):
    return (-(-M // NS) + 7) // 8 * 8


@functools.lru_cache(maxsize=None)
def _gather_maxpos(B, M, D, UNROLL=4):
    b_per_w = B // NW
    n_chunks = B // LANES
    rng = _rng(M)
    m_pad = rng * NS
    mesh = plsc.VectorSubcoreMesh(core_axis_name="c", subcore_axis_name="s")

    @functools.partial(
        pl.kernel,
        mesh=mesh,
        out_type=(
            jax.ShapeDtypeStruct((B, D), jnp.float32),
            jax.ShapeDtypeStruct((m_pad,), jnp.int32),
            jax.ShapeDtypeStruct((m_pad,), jnp.float32),
        ),
        compiler_params=pltpu.CompilerParams(needs_layout_passes=False),
        scratch_types=[
            pltpu.VMEM((B,), jnp.int32),
            pltpu.VMEM((B,), jnp.float32),
            pltpu.VMEM((rng,), jnp.int32),
            pltpu.VMEM((rng,), jnp.float32),
            pltpu.VMEM((rng,), jnp.float32),
            pltpu.VMEM((b_per_w, D), jnp.float32),
            pltpu.SemaphoreType.DMA,
            pltpu.SemaphoreType.DMA,
        ],
    )
    def gather_k(ids_hbm, ts_hbm, lu_hbm, table_hbm, h_hbm, mp_hbm, luo_hbm,
                 ids_all, ts_all, mp_loc, ts_loc, lu_v, rows_v, sem, sem2):
        s = lax.axis_index("s")
        wid = _wid()
        base = wid * b_per_w
        lo = s * rng
        pltpu.sync_copy(ids_hbm, ids_all)
        g = pltpu.async_copy(
            table_hbm.at[ids_all.at[pl.ds(base, b_per_w)]], rows_v, sem)
        g2 = pltpu.async_copy(ts_hbm, ts_all, sem2)
        g3 = pltpu.async_copy(lu_hbm.at[pl.ds(lo, rng)], lu_v, sem2)

        lane = lax.iota(jnp.int32, LANES)
        neg1 = jnp.full((LANES,), -1, jnp.int32)

        def clr(r, carry):
            mp_loc[pl.ds(r * LANES, LANES)] = neg1
            return carry

        lax.fori_loop(0, rng // LANES, clr, 0)
        g2.wait()

        def body(k, carry):
            ks = [k * UNROLL + u for u in range(UNROLL)]
            idxs = [ids_all[pl.ds(kk * LANES, LANES)] for kk in ks]
            tss = [ts_all[pl.ds(kk * LANES, LANES)] for kk in ks]
            lasts = [plsc.scan_count(idx)[1] for idx in idxs]
            for kk, idx, ts, last_m in zip(ks, idxs, tss, lasts):
                pos = jnp.full((LANES,), kk * LANES, jnp.int32) + lane
                m = (idx >= lo) & (idx < lo + rng) & last_m
                loc = jnp.clip(idx - lo, 0, rng - 1)
                plsc.store_scatter(mp_loc, [loc], pos, mask=m)
                plsc.store_scatter(ts_loc, [loc], ts, mask=m)
            return carry

        lax.fori_loop(0, n_chunks // UNROLL, body, 0)
        pltpu.sync_copy(mp_loc, mp_hbm.at[pl.ds(lo, rng)])

        g3.wait()

        def mrg(r, carry):
            sl = pl.ds(r * LANES, LANES)
            touched = mp_loc[sl] >= 0
            lu_v[sl] = jnp.where(touched, ts_loc[sl], lu_v[sl])
            return carry

        lax.fori_loop(0, rng // LANES, mrg, 0)
        pltpu.sync_copy(lu_v, luo_hbm.at[pl.ds(lo, rng)])
        g.wait()
        pltpu.sync_copy(rows_v, h_hbm.at[pl.ds(base, b_per_w)])

    return gather_k


@functools.lru_cache(maxsize=None)
def _gru(B, D, BLK=2048):
    grid = (B // BLK,)

    def body(msg_ref, h_ref, wih_ref, whh_ref, bih_ref, bhh_ref, rows_ref):
        gi = jnp.dot(msg_ref[...], wih_ref[...],
                     preferred_element_type=jnp.float32) + bih_ref[...]
        gh = jnp.dot(h_ref[...], whh_ref[...],
                     preferred_element_type=jnp.float32) + bhh_ref[...]
        i_r, i_z, i_n = gi[:, :D], gi[:, D:2 * D], gi[:, 2 * D:]
        h_r, h_z, h_n = gh[:, :D], gh[:, D:2 * D], gh[:, 2 * D:]
        r = jax.nn.sigmoid(i_r + h_r)
        z = jax.nn.sigmoid(i_z + h_z)
        n = jnp.tanh(i_n + r * h_n)
        h = h_ref[...]
        rows_ref[...] = (1.0 - z) * n + z * h

    return pl.pallas_call(
        body,
        grid=grid,
        in_specs=[
            pl.BlockSpec((BLK, D), lambda i: (i, 0)),
            pl.BlockSpec((BLK, D), lambda i: (i, 0)),
            pl.BlockSpec((D, 3 * D), lambda i: (0, 0)),
            pl.BlockSpec((D, 3 * D), lambda i: (0, 0)),
            pl.BlockSpec((1, 3 * D), lambda i: (0, 0)),
            pl.BlockSpec((1, 3 * D), lambda i: (0, 0)),
        ],
        out_specs=pl.BlockSpec((BLK, D), lambda i: (i, 0)),
        out_shape=jax.ShapeDtypeStruct((B, D), jnp.float32),
    )


@functools.lru_cache(maxsize=None)
def _copy(M, D, NSTEP=8):
    cpb = (-(-M // NSTEP) + 7) // 8 * 8

    def body(mem_ref, cp_ref):
        cp_ref[...] = mem_ref[...]

    return pl.pallas_call(
        body,
        grid=(NSTEP,),
        in_specs=[pl.BlockSpec((cpb, D), lambda i: (i, 0))],
        out_specs=pl.BlockSpec((cpb, D), lambda i: (i, 0)),
        out_shape=jax.ShapeDtypeStruct((M, D), jnp.float32),
    )


@functools.lru_cache(maxsize=None)
def _scatter(B, M, D, NB=4):
    b_per_w = B // NW
    blk = b_per_w // NB
    rng = _rng(M)
    mesh = plsc.VectorSubcoreMesh(core_axis_name="c", subcore_axis_name="s")

    @functools.partial(
        pl.kernel,
        mesh=mesh,
        out_type=(),
        compiler_params=pltpu.CompilerParams(needs_layout_passes=False),
        scratch_types=[
            pltpu.VMEM((NB, blk), jnp.int32),
            pltpu.VMEM((NB, blk), jnp.int32),
            pltpu.VMEM((NB, blk, D), jnp.float32),
            [pltpu.SemaphoreType.DMA] * NB,
            [pltpu.SemaphoreType.DMA] * NB,
        ],
    )
    def scatter_k(ids_hbm, mp_hbm, rows_hbm, mem_ref,
                  idx_b, mp_b, rows_b, gsems, ssems):
        wid = _wid()
        base = wid * b_per_w
        for j in range(NB):
            pltpu.sync_copy(ids_hbm.at[pl.ds(base + j * blk, blk)],
                            idx_b.at[j])
        mp_g = [pltpu.async_copy(mp_hbm.at[idx_b.at[j]], mp_b.at[j], gsems[j])
                for j in range(NB)]
        row_g, outs = [], []
        for j in range(NB):
            mp_g[j].wait()
            row_g.append(pltpu.async_copy(
                rows_hbm.at[mp_b.at[j]], rows_b.at[j], gsems[j]))
        for j in range(NB):
            row_g[j].wait()
            outs.append(pltpu.async_copy(
                rows_b.at[j], mem_ref.at[idx_b.at[j]], ssems[j]))
        for cp in outs:
            cp.wait()

    return scatter_k


def kernel(memory, last_update, unique_node_ids, unique_messages, timestamps,
           W_ih, W_hh, b_ih, b_hh):
    M, D = memory.shape
    B = unique_node_ids.shape[0]
    ids = unique_node_ids.astype(jnp.int32)

    m_pad = _rng(M) * NS
    lu_pad = jnp.pad(last_update, (0, m_pad - M))
    h, maxpos, lu_full = _gather_maxpos(B, M, D)(ids, timestamps, lu_pad,
                                                 memory)
    mem_copy = _copy(M, D)(memory)
    rows = _gru(B, D)(unique_messages, h, W_ih.T, W_hh.T,
                      b_ih.reshape(1, -1), b_hh.reshape(1, -1))

    mem_ref = jax.new_ref(mem_copy)
    _scatter(B, M, D)(ids, maxpos, rows, mem_ref)
    return mem_ref[...], lu_full[:M]


# speedup vs baseline: 2.7217x; 1.0128x over previous
import functools

import jax
import jax.numpy as jnp
from jax import lax
from jax.experimental import pallas as pl
from jax.experimental.pallas import tpu as pltpu
from jax.experimental.pallas import tpu_sc as plsc

NC = 2
NS = 16
NW = NC * NS
LANES = 16


def _wid():
    return lax.axis_index("s") * NC + lax.axis_index("c")


def _rng(M):
    return (-(-M // NS) + 7) // 8 * 8


@functools.lru_cache(maxsize=None)
def _gather_maxpos(B, M, D, UNROLL=4):
    b_per_w = B // NW
    n_chunks = B // LANES
    rng = _rng(M)
    m_pad = rng * NS
    mesh = plsc.VectorSubcoreMesh(core_axis_name="c", subcore_axis_name="s")

    @functools.partial(
        pl.kernel,
        mesh=mesh,
        out_type=(
            jax.ShapeDtypeStruct((B, D), jnp.float32),
            jax.ShapeDtypeStruct((m_pad,), jnp.int32),
            jax.ShapeDtypeStruct((m_pad,), jnp.float32),
        ),
        compiler_params=pltpu.CompilerParams(needs_layout_passes=False),
        scratch_types=[
            pltpu.VMEM((B,), jnp.int32),
            pltpu.VMEM((B,), jnp.float32),
            pltpu.VMEM((rng,), jnp.int32),
            pltpu.VMEM((rng,), jnp.float32),
            pltpu.VMEM((rng,), jnp.float32),
            pltpu.VMEM((b_per_w, D), jnp.float32),
            pltpu.SemaphoreType.DMA,
            pltpu.SemaphoreType.DMA,
        ],
    )
    def gather_k(ids_hbm, ts_hbm, lu_hbm, table_hbm, h_hbm, mp_hbm, luo_hbm,
                 ids_all, ts_all, mp_loc, ts_loc, lu_v, rows_v,
                 sem, sem2):
        s = lax.axis_index("s")
        wid = _wid()
        base = wid * b_per_w
        lo = s * rng

        sl = B // NS
        rot = []
        for j in range(NS):
            off = ((s + j) % NS) * sl
            st = pl.ds(off, sl)
            rot.append(pltpu.async_copy(ids_hbm.at[st], ids_all.at[st], sem))
            rot.append(pltpu.async_copy(ts_hbm.at[st], ts_all.at[st], sem2))
        for cp in rot:
            cp.wait()
        g = pltpu.async_copy(
            table_hbm.at[ids_all.at[pl.ds(base, b_per_w)]], rows_v, sem)
        g3 = pltpu.async_copy(lu_hbm.at[pl.ds(lo, rng)], lu_v, sem2)

        lane = lax.iota(jnp.int32, LANES)
        neg1 = jnp.full((LANES,), -1, jnp.int32)

        def clr(r, carry):
            mp_loc[pl.ds(r * LANES, LANES)] = neg1
            return carry

        lax.fori_loop(0, rng // LANES, clr, 0)

        def body(k, carry):
            ks = [k * UNROLL + u for u in range(UNROLL)]
            idxs = [ids_all[pl.ds(kk * LANES, LANES)] for kk in ks]
            tss = [ts_all[pl.ds(kk * LANES, LANES)] for kk in ks]
            lasts = [plsc.scan_count(idx)[1] for idx in idxs]
            for kk, idx, ts, last_m in zip(ks, idxs, tss, lasts):
                pos = jnp.full((LANES,), kk * LANES, jnp.int32) + lane
                m = (idx >= lo) & (idx < lo + rng) & last_m
                loc = jnp.clip(idx - lo, 0, rng - 1)
                plsc.store_scatter(mp_loc, [loc], pos, mask=m)
                plsc.store_scatter(ts_loc, [loc], ts, mask=m)
            return carry

        lax.fori_loop(0, n_chunks // UNROLL, body, 0)
        pltpu.sync_copy(mp_loc, mp_hbm.at[pl.ds(lo, rng)])

        g3.wait()

        def mrg(r, carry):
            sl = pl.ds(r * LANES, LANES)
            touched = mp_loc[sl] >= 0
            lu_v[sl] = jnp.where(touched, ts_loc[sl], lu_v[sl])
            return carry

        lax.fori_loop(0, rng // LANES, mrg, 0)
        pltpu.sync_copy(lu_v, luo_hbm.at[pl.ds(lo, rng)])
        g.wait()
        pltpu.sync_copy(rows_v, h_hbm.at[pl.ds(base, b_per_w)])

    return gather_k


@functools.lru_cache(maxsize=None)
def _gru(B, D, BLK=2048):
    grid = (B // BLK,)

    def body(msg_ref, h_ref, wih_ref, whh_ref, bih_ref, bhh_ref, rows_ref):
        gi = jnp.dot(msg_ref[...], wih_ref[...],
                     preferred_element_type=jnp.float32) + bih_ref[...]
        gh = jnp.dot(h_ref[...], whh_ref[...],
                     preferred_element_type=jnp.float32) + bhh_ref[...]
        i_r, i_z, i_n = gi[:, :D], gi[:, D:2 * D], gi[:, 2 * D:]
        h_r, h_z, h_n = gh[:, :D], gh[:, D:2 * D], gh[:, 2 * D:]
        r = jax.nn.sigmoid(i_r + h_r)
        z = jax.nn.sigmoid(i_z + h_z)
        n = jnp.tanh(i_n + r * h_n)
        h = h_ref[...]
        rows_ref[...] = (1.0 - z) * n + z * h

    return pl.pallas_call(
        body,
        grid=grid,
        in_specs=[
            pl.BlockSpec((BLK, D), lambda i: (i, 0)),
            pl.BlockSpec((BLK, D), lambda i: (i, 0)),
            pl.BlockSpec((D, 3 * D), lambda i: (0, 0)),
            pl.BlockSpec((D, 3 * D), lambda i: (0, 0)),
            pl.BlockSpec((1, 3 * D), lambda i: (0, 0)),
            pl.BlockSpec((1, 3 * D), lambda i: (0, 0)),
        ],
        out_specs=pl.BlockSpec((BLK, D), lambda i: (i, 0)),
        out_shape=jax.ShapeDtypeStruct((B, D), jnp.float32),
    )


@functools.lru_cache(maxsize=None)
def _copy(M, D, NSTEP=8):
    cpb = (-(-M // NSTEP) + 7) // 8 * 8

    def body(mem_ref, cp_ref):
        cp_ref[...] = mem_ref[...]

    return pl.pallas_call(
        body,
        grid=(NSTEP,),
        in_specs=[pl.BlockSpec((cpb, D), lambda i: (i, 0))],
        out_specs=pl.BlockSpec((cpb, D), lambda i: (i, 0)),
        out_shape=jax.ShapeDtypeStruct((M, D), jnp.float32),
    )


@functools.lru_cache(maxsize=None)
def _scatter(B, M, D, NB=4):
    b_per_w = B // NW
    blk = b_per_w // NB
    rng = _rng(M)
    mesh = plsc.VectorSubcoreMesh(core_axis_name="c", subcore_axis_name="s")

    @functools.partial(
        pl.kernel,
        mesh=mesh,
        out_type=(),
        compiler_params=pltpu.CompilerParams(needs_layout_passes=False),
        scratch_types=[
            pltpu.VMEM((NB, blk), jnp.int32),
            pltpu.VMEM((NB, blk), jnp.int32),
            pltpu.VMEM((NB, blk, D), jnp.float32),
            [pltpu.SemaphoreType.DMA] * NB,
            [pltpu.SemaphoreType.DMA] * NB,
        ],
    )
    def scatter_k(ids_hbm, mp_hbm, rows_hbm, mem_ref,
                  idx_b, mp_b, rows_b, gsems, ssems):
        wid = _wid()
        base = wid * b_per_w
        for j in range(NB):
            pltpu.sync_copy(ids_hbm.at[pl.ds(base + j * blk, blk)],
                            idx_b.at[j])
        mp_g = [pltpu.async_copy(mp_hbm.at[idx_b.at[j]], mp_b.at[j], gsems[j])
                for j in range(NB)]
        row_g, outs = [], []
        for j in range(NB):
            mp_g[j].wait()
            row_g.append(pltpu.async_copy(
                rows_hbm.at[mp_b.at[j]], rows_b.at[j], gsems[j]))
        for j in range(NB):
            row_g[j].wait()
            outs.append(pltpu.async_copy(
                rows_b.at[j], mem_ref.at[idx_b.at[j]], ssems[j]))
        for cp in outs:
            cp.wait()

    return scatter_k


def kernel(memory, last_update, unique_node_ids, unique_messages, timestamps,
           W_ih, W_hh, b_ih, b_hh):
    M, D = memory.shape
    B = unique_node_ids.shape[0]
    ids = unique_node_ids.astype(jnp.int32)

    m_pad = _rng(M) * NS
    lu_pad = jnp.pad(last_update, (0, m_pad - M))
    h, maxpos, lu_full = _gather_maxpos(B, M, D)(ids, timestamps, lu_pad,
                                                 memory)
    mem_copy = _copy(M, D)(memory)
    rows = _gru(B, D)(unique_messages, h, W_ih.T, W_hh.T,
                      b_ih.reshape(1, -1), b_hh.reshape(1, -1))

    mem_ref = jax.new_ref(mem_copy)
    _scatter(B, M, D)(ids, maxpos, rows, mem_ref)
    return mem_ref[...], lu_full[:M]


# speedup vs baseline: 2.7467x; 1.0092x over previous
import functools

import jax
import jax.numpy as jnp
from jax import lax
from jax.experimental import pallas as pl
from jax.experimental.pallas import tpu as pltpu
from jax.experimental.pallas import tpu_sc as plsc

NC = 2
NS = 16
NW = NC * NS
LANES = 16


def _wid():
    return lax.axis_index("s") * NC + lax.axis_index("c")


def _rng(M):
    return (-(-M // NS) + 63) // 64 * 64


@functools.lru_cache(maxsize=None)
def _gather_maxpos(B, M, D, UNROLL=4):
    b_per_w = B // NW
    n_chunks = B // LANES
    rng = _rng(M)
    m_pad = rng * NS
    mesh = plsc.VectorSubcoreMesh(core_axis_name="c", subcore_axis_name="s")

    @functools.partial(
        pl.kernel,
        mesh=mesh,
        out_type=(
            jax.ShapeDtypeStruct((B, D), jnp.float32),
            jax.ShapeDtypeStruct((m_pad,), jnp.int32),
            jax.ShapeDtypeStruct((m_pad,), jnp.float32),
        ),
        compiler_params=pltpu.CompilerParams(needs_layout_passes=False),
        scratch_types=[
            pltpu.VMEM((B,), jnp.int32),
            pltpu.VMEM((B,), jnp.float32),
            pltpu.VMEM((rng,), jnp.int32),
            pltpu.VMEM((rng,), jnp.float32),
            pltpu.VMEM((rng,), jnp.float32),
            pltpu.VMEM((b_per_w, D), jnp.float32),
            pltpu.SemaphoreType.DMA,
            pltpu.SemaphoreType.DMA,
        ],
    )
    def gather_k(ids_hbm, ts_hbm, lu_hbm, table_hbm, h_hbm, mp_hbm, luo_hbm,
                 ids_all, ts_all, mp_loc, ts_loc, lu_v, rows_v,
                 sem, sem2):
        s = lax.axis_index("s")
        wid = _wid()
        base = wid * b_per_w
        lo = s * rng

        sl = B // NS
        rot = []
        for j in range(NS):
            off = ((s + j) % NS) * sl
            st = pl.ds(off, sl)
            rot.append(pltpu.async_copy(ids_hbm.at[st], ids_all.at[st], sem))
            rot.append(pltpu.async_copy(ts_hbm.at[st], ts_all.at[st], sem2))
        for cp in rot:
            cp.wait()
        g = pltpu.async_copy(
            table_hbm.at[ids_all.at[pl.ds(base, b_per_w)]], rows_v, sem)
        g3 = pltpu.async_copy(lu_hbm.at[pl.ds(lo, rng)], lu_v, sem2)

        lane = lax.iota(jnp.int32, LANES)
        neg1 = jnp.full((LANES,), -1, jnp.int32)

        def clr(r, carry):
            for u in range(4):
                mp_loc[pl.ds((r * 4 + u) * LANES, LANES)] = neg1
            return carry

        lax.fori_loop(0, rng // (4 * LANES), clr, 0)

        def body(k, carry):
            ks = [k * UNROLL + u for u in range(UNROLL)]
            idxs = [ids_all[pl.ds(kk * LANES, LANES)] for kk in ks]
            tss = [ts_all[pl.ds(kk * LANES, LANES)] for kk in ks]
            lasts = [plsc.scan_count(idx)[1] for idx in idxs]
            for kk, idx, ts, last_m in zip(ks, idxs, tss, lasts):
                pos = jnp.full((LANES,), kk * LANES, jnp.int32) + lane
                m = (idx >= lo) & (idx < lo + rng) & last_m
                loc = jnp.clip(idx - lo, 0, rng - 1)
                plsc.store_scatter(mp_loc, [loc], pos, mask=m)
                plsc.store_scatter(ts_loc, [loc], ts, mask=m)
            return carry

        lax.fori_loop(0, n_chunks // UNROLL, body, 0)
        pltpu.sync_copy(mp_loc, mp_hbm.at[pl.ds(lo, rng)])

        g3.wait()

        def mrg(r, carry):
            for u in range(4):
                slu = pl.ds((r * 4 + u) * LANES, LANES)
                touched = mp_loc[slu] >= 0
                lu_v[slu] = jnp.where(touched, ts_loc[slu], lu_v[slu])
            return carry

        lax.fori_loop(0, rng // (4 * LANES), mrg, 0)
        pltpu.sync_copy(lu_v, luo_hbm.at[pl.ds(lo, rng)])
        g.wait()
        pltpu.sync_copy(rows_v, h_hbm.at[pl.ds(base, b_per_w)])

    return gather_k


@functools.lru_cache(maxsize=None)
def _gru(B, D, BLK=2048):
    grid = (B // BLK,)

    def body(msg_ref, h_ref, wih_ref, whh_ref, bih_ref, bhh_ref, rows_ref):
        gi = jnp.dot(msg_ref[...], wih_ref[...],
                     preferred_element_type=jnp.float32) + bih_ref[...]
        gh = jnp.dot(h_ref[...], whh_ref[...],
                     preferred_element_type=jnp.float32) + bhh_ref[...]
        i_r, i_z, i_n = gi[:, :D], gi[:, D:2 * D], gi[:, 2 * D:]
        h_r, h_z, h_n = gh[:, :D], gh[:, D:2 * D], gh[:, 2 * D:]
        r = jax.nn.sigmoid(i_r + h_r)
        z = jax.nn.sigmoid(i_z + h_z)
        n = jnp.tanh(i_n + r * h_n)
        h = h_ref[...]
        rows_ref[...] = (1.0 - z) * n + z * h

    return pl.pallas_call(
        body,
        grid=grid,
        in_specs=[
            pl.BlockSpec((BLK, D), lambda i: (i, 0)),
            pl.BlockSpec((BLK, D), lambda i: (i, 0)),
            pl.BlockSpec((D, 3 * D), lambda i: (0, 0)),
            pl.BlockSpec((D, 3 * D), lambda i: (0, 0)),
            pl.BlockSpec((1, 3 * D), lambda i: (0, 0)),
            pl.BlockSpec((1, 3 * D), lambda i: (0, 0)),
        ],
        out_specs=pl.BlockSpec((BLK, D), lambda i: (i, 0)),
        out_shape=jax.ShapeDtypeStruct((B, D), jnp.float32),
    )


@functools.lru_cache(maxsize=None)
def _copy(M, D, NSTEP=8):
    cpb = (-(-M // NSTEP) + 7) // 8 * 8

    def body(mem_ref, cp_ref):
        cp_ref[...] = mem_ref[...]

    return pl.pallas_call(
        body,
        grid=(NSTEP,),
        in_specs=[pl.BlockSpec((cpb, D), lambda i: (i, 0))],
        out_specs=pl.BlockSpec((cpb, D), lambda i: (i, 0)),
        out_shape=jax.ShapeDtypeStruct((M, D), jnp.float32),
    )


@functools.lru_cache(maxsize=None)
def _scatter(B, M, D, NB=4):
    b_per_w = B // NW
    blk = b_per_w // NB
    rng = _rng(M)
    mesh = plsc.VectorSubcoreMesh(core_axis_name="c", subcore_axis_name="s")

    @functools.partial(
        pl.kernel,
        mesh=mesh,
        out_type=(),
        compiler_params=pltpu.CompilerParams(needs_layout_passes=False),
        scratch_types=[
            pltpu.VMEM((NB, blk), jnp.int32),
            pltpu.VMEM((NB, blk), jnp.int32),
            pltpu.VMEM((NB, blk, D), jnp.float32),
            [pltpu.SemaphoreType.DMA] * NB,
            [pltpu.SemaphoreType.DMA] * NB,
        ],
    )
    def scatter_k(ids_hbm, mp_hbm, rows_hbm, mem_ref,
                  idx_b, mp_b, rows_b, gsems, ssems):
        wid = _wid()
        base = wid * b_per_w
        for j in range(NB):
            pltpu.sync_copy(ids_hbm.at[pl.ds(base + j * blk, blk)],
                            idx_b.at[j])
        mp_g = [pltpu.async_copy(mp_hbm.at[idx_b.at[j]], mp_b.at[j], gsems[j])
                for j in range(NB)]
        row_g, outs = [], []
        for j in range(NB):
            mp_g[j].wait()
            row_g.append(pltpu.async_copy(
                rows_hbm.at[mp_b.at[j]], rows_b.at[j], gsems[j]))
        for j in range(NB):
            row_g[j].wait()
            outs.append(pltpu.async_copy(
                rows_b.at[j], mem_ref.at[idx_b.at[j]], ssems[j]))
        for cp in outs:
            cp.wait()

    return scatter_k


def kernel(memory, last_update, unique_node_ids, unique_messages, timestamps,
           W_ih, W_hh, b_ih, b_hh):
    M, D = memory.shape
    B = unique_node_ids.shape[0]
    ids = unique_node_ids.astype(jnp.int32)

    m_pad = _rng(M) * NS
    lu_pad = jnp.pad(last_update, (0, m_pad - M))
    h, maxpos, lu_full = _gather_maxpos(B, M, D)(ids, timestamps, lu_pad,
                                                 memory)
    mem_copy = _copy(M, D)(memory)
    rows = _gru(B, D)(unique_messages, h, W_ih.T, W_hh.T,
                      b_ih.reshape(1, -1), b_hh.reshape(1, -1))

    mem_ref = jax.new_ref(mem_copy)
    _scatter(B, M, D)(ids, maxpos, rows, mem_ref)
    return mem_ref[...], lu_full[:M]
